# Initial kernel scaffold; baseline (speedup 1.0000x reference)
import jax
import jax.numpy as jnp
from jax.experimental import pallas as pl


def kernel(image_feature, point_cloud_feature, prev_image_feature, prev_point_cloud_feature, rel_dist_mask, prev_spatial, img_W1, img_b1, img_W2, img_b2, img_W3, img_b3, img_g, img_be, pc_W1, pc_b1, pc_W2, pc_b2, pc_W3, pc_b3, pc_g, pc_be, fus_W1, fus_b1, fus_W2, fus_b2, fus_g, fus_be):
    raise NotImplementedError("write your pallas kernel here")


# speedup vs baseline: 7.9294x; 7.9294x over previous
import functools

import jax
import jax.numpy as jnp
from jax import lax
from jax.experimental import pallas as pl
from jax.experimental.pallas import tpu as pltpu
from jax.experimental.pallas import tpu_sc as plsc

B, N = 4, 128
IMG_D = 256
PC_D = 256
VIS_D = 256
THRESH_VAL = 0.0
NEG = -2.0
HIGHEST = lax.Precision.HIGHEST
L = 16
NCHUNK = N // L


def _dot_t(x, w):
    return lax.dot_general(x, w, (((1,), (1,)), ((), ())), precision=HIGHEST,
                           preferred_element_type=jnp.float32)


def _sims_body(img_ref, pc_ref, pimg_ref, ppc_ref, mask_ref, s_ref):
    img = img_ref[0]
    pc = pc_ref[0]
    pimg = pimg_ref[0]
    ppc = ppc_ref[0]
    ss_f = jnp.sum(img * img, axis=1, keepdims=True) + jnp.sum(pc * pc, axis=1, keepdims=True)
    ss_p = jnp.sum(pimg * pimg, axis=1, keepdims=True) + jnp.sum(ppc * ppc, axis=1, keepdims=True)
    den_f = jnp.maximum(jnp.sqrt(ss_f), 1e-8)
    den_p = jnp.maximum(jnp.sqrt(ss_p), 1e-8)
    fn_img = img / den_f
    fn_pc = pc / den_f
    pn_img = pimg / den_p
    pn_pc = ppc / den_p
    sims = _dot_t(fn_img, pn_img) + _dot_t(fn_pc, pn_pc)
    s_ref[0] = jnp.where(mask_ref[0] > 0.0, sims, NEG)


def _masked_sims(img, pc, pimg, ppc, mask_f):
    feat_spec = pl.BlockSpec((1, N, IMG_D), lambda b: (b, 0, 0))
    mat_spec = pl.BlockSpec((1, N, N), lambda b: (b, 0, 0))
    return pl.pallas_call(
        _sims_body,
        grid=(B,),
        in_specs=[feat_spec, feat_spec, feat_spec, feat_spec, mat_spec],
        out_specs=mat_spec,
        out_shape=jax.ShapeDtypeStruct((B, N, N), jnp.float32),
    )(img, pc, pimg, ppc, mask_f)


def _greedy_body(s_hbm, idx_hbm, s_v, vis_v, idx_v, rot_f, rot_i):
    info = plsc.get_sparse_core_info()
    wid = lax.axis_index("s") * info.num_cores + lax.axis_index("c")

    def rotmax_f(v):
        for k in (8, 4, 2, 1):
            rot_f[pl.ds(0, L)] = v
            rot_f[pl.ds(L, L)] = v
            v = jnp.maximum(v, rot_f[pl.ds(k, L)])
        return v

    def rotmin_i(v):
        for k in (8, 4, 2, 1):
            rot_i[pl.ds(0, L)] = v
            rot_i[pl.ds(L, L)] = v
            v = jnp.minimum(v, rot_i[pl.ds(k, L)])
        return v

    @pl.when(wid < B)
    def _():
        pltpu.sync_copy(s_hbm.at[wid], s_v)
        zeros = jnp.zeros((L,), jnp.float32)
        neg1 = jnp.full((L,), -1, jnp.int32)
        for c in range(NCHUNK):
            vis_v[pl.ds(c * L, L)] = zeros
            idx_v[pl.ds(c * L, L)] = neg1
        iota = lax.iota(jnp.int32, L)
        big = jnp.int32(32767)

        def body(i, carry):
            effs = []
            vm = jnp.full((L,), -3.0, jnp.float32)
            for c in range(NCHUNK):
                sv = s_v[i, pl.ds(c * L, L)]
                vis = vis_v[pl.ds(c * L, L)]
                eff = jnp.where(vis > 0.0, NEG, sv)
                effs.append(eff)
                vm = jnp.maximum(vm, eff)
            m_v = rotmax_f(vm)
            cand = jnp.full((L,), big, jnp.int32)
            for c in range(NCHUNK):
                cand = jnp.minimum(cand, jnp.where(effs[c] == m_v, iota + c * L, big))
            j_v = rotmin_i(cand)
            ok_v = m_v >= THRESH_VAL
            ibase = (i >> 4) << 4
            ioff = i & 15
            old_idx = idx_v[pl.ds(ibase, L)]
            idx_v[pl.ds(ibase, L)] = jnp.where(
                iota == ioff, jnp.where(ok_v, j_v, neg1), old_idx)
            for c in range(NCHUNK):
                vis = vis_v[pl.ds(c * L, L)]
                vis_v[pl.ds(c * L, L)] = jnp.where(
                    ((iota + c * L) == j_v) & ok_v, 1.0, vis)
            return carry

        lax.fori_loop(0, N, body, 0)
        pltpu.sync_copy(idx_v, idx_hbm.at[wid])


def _greedy_match(s0):
    mesh = plsc.VectorSubcoreMesh(core_axis_name="c", subcore_axis_name="s")
    return pl.kernel(
        _greedy_body,
        out_type=jax.ShapeDtypeStruct((B, N), jnp.int32),
        mesh=mesh,
        scratch_types=[
            pltpu.VMEM((N, N), jnp.float32),
            pltpu.VMEM((N,), jnp.float32),
            pltpu.VMEM((N,), jnp.int32),
            pltpu.VMEM((2 * L,), jnp.float32),
            pltpu.VMEM((2 * L,), jnp.int32),
        ],
    )(s0)


def _ln_rows(x, g, b):
    m = jnp.mean(x, axis=-1, keepdims=True)
    v = jnp.mean((x - m) ** 2, axis=-1, keepdims=True)
    return (x - m) / jnp.sqrt(v + 1e-5) * g + b


def _mlp_body(idx_ref, img_ref, pc_ref, pimg_ref, ppc_ref, pspat_ref,
              iW1_ref, ib1_ref, iW2_ref, ib2_ref, iW3_ref, ib3_ref, ig_ref, ibe_ref,
              pW1_ref, pb1_ref, pW2_ref, pb2_ref, pW3_ref, pb3_ref, pg_ref, pbe_ref,
              fW1_ref, fb1_ref, fW2_ref, fb2_ref, fg_ref, fbe_ref,
              vis_ref, nps_ref):
    idx = idx_ref[0]
    img = img_ref[0]
    pc = pc_ref[0]
    pimg = pimg_ref[0]
    ppc = ppc_ref[0]
    pspat = pspat_ref[0]
    jrow = lax.broadcasted_iota(jnp.int32, (N, N), 0)
    E = ((jrow == idx) & (idx >= 0)).astype(jnp.float32)

    def gather(prev):
        return lax.dot_general(E, prev, (((0,), (0,)), ((), ())), precision=HIGHEST,
                               preferred_element_type=jnp.float32)

    prev_img_g = gather(pimg)
    prev_pc_g = gather(ppc)
    nps_ref[0] = gather(pspat)

    iW1 = iW1_ref[...]
    h = jnp.maximum(_dot_t(img, iW1[:, :IMG_D]) + _dot_t(prev_img_g, iW1[:, IMG_D:])
                    + ib1_ref[...], 0.0)
    h = _dot_t(h, iW2_ref[...]) + ib2_ref[...]
    h = _dot_t(h, iW3_ref[...]) + ib3_ref[...]
    img_o = _ln_rows(h, ig_ref[...], ibe_ref[...])

    pW1 = pW1_ref[...]
    h = jnp.maximum(_dot_t(pc, pW1[:, :PC_D]) + _dot_t(prev_pc_g, pW1[:, PC_D:])
                    + pb1_ref[...], 0.0)
    h = _dot_t(h, pW2_ref[...]) + pb2_ref[...]
    h = _dot_t(h, pW3_ref[...]) + pb3_ref[...]
    pc_o = _ln_rows(h, pg_ref[...], pbe_ref[...])

    fW1 = fW1_ref[...]
    h = jnp.maximum(_dot_t(img_o, fW1[:, :VIS_D]) + _dot_t(pc_o, fW1[:, VIS_D:])
                    + fb1_ref[...], 0.0)
    h = _dot_t(h, fW2_ref[...]) + fb2_ref[...]
    vis_ref[0] = _ln_rows(h, fg_ref[...], fbe_ref[...])


def _fusion_mlp(idx3, img, pc, pimg, ppc, pspat_pad, weights):
    feat_spec = pl.BlockSpec((1, N, IMG_D), lambda b: (b, 0, 0))
    mat_spec = pl.BlockSpec((1, N, 128), lambda b: (b, 0, 0))
    idx_spec = pl.BlockSpec((1, 1, N), lambda b: (b, 0, 0))
    full = lambda arr: pl.BlockSpec(arr.shape, lambda b: tuple(0 for _ in arr.shape))
    w_specs = [full(w) for w in weights]
    return pl.pallas_call(
        _mlp_body,
        grid=(B,),
        in_specs=[idx_spec, feat_spec, feat_spec, feat_spec, feat_spec, mat_spec] + w_specs,
        out_specs=[pl.BlockSpec((1, N, VIS_D), lambda b: (b, 0, 0)), mat_spec],
        out_shape=[jax.ShapeDtypeStruct((B, N, VIS_D), jnp.float32),
                   jax.ShapeDtypeStruct((B, N, 128), jnp.float32)],
    )(idx3, img, pc, pimg, ppc, pspat_pad, *weights)


def kernel(image_feature, point_cloud_feature, prev_image_feature, prev_point_cloud_feature, rel_dist_mask, prev_spatial, img_W1, img_b1, img_W2, img_b2, img_W3, img_b3, img_g, img_be, pc_W1, pc_b1, pc_W2, pc_b2, pc_W3, pc_b3, pc_g, pc_be, fus_W1, fus_b1, fus_W2, fus_b2, fus_g, fus_be):
    mask_f = rel_dist_mask.astype(jnp.float32)
    s0 = _masked_sims(image_feature, point_cloud_feature,
                      prev_image_feature, prev_point_cloud_feature, mask_f)
    idx = _greedy_match(s0)
    idx3 = idx.reshape(B, 1, N)
    pspat_pad = jnp.pad(prev_spatial, ((0, 0), (0, 0), (0, 128 - prev_spatial.shape[2])))
    r2 = lambda v: v.reshape(1, -1)
    weights = (img_W1, r2(img_b1), img_W2, r2(img_b2), img_W3, r2(img_b3), r2(img_g), r2(img_be),
               pc_W1, r2(pc_b1), pc_W2, r2(pc_b2), pc_W3, r2(pc_b3), r2(pc_g), r2(pc_be),
               fus_W1, r2(fus_b1), fus_W2, r2(fus_b2), r2(fus_g), r2(fus_be))
    vis, nps_pad = _fusion_mlp(idx3, image_feature, point_cloud_feature,
                               prev_image_feature, prev_point_cloud_feature,
                               pspat_pad, weights)
    return (vis, nps_pad[:, :, :7])


# speedup vs baseline: 7.9876x; 1.0073x over previous
import functools

import jax
import jax.numpy as jnp
from jax import lax
from jax.experimental import pallas as pl
from jax.experimental.pallas import tpu as pltpu
from jax.experimental.pallas import tpu_sc as plsc

B, N = 4, 128
IMG_D = 256
PC_D = 256
VIS_D = 256
THRESH_VAL = 0.0
NEG = -2.0
HIGHEST = lax.Precision.HIGHEST
L = 16
NCHUNK = N // L


def _dot_t(x, w):
    return lax.dot_general(x, w, (((1,), (1,)), ((), ())), precision=HIGHEST,
                           preferred_element_type=jnp.float32)


def _sims_body(img_ref, pc_ref, pimg_ref, ppc_ref, mask_ref, s_ref):
    img = img_ref[0]
    pc = pc_ref[0]
    pimg = pimg_ref[0]
    ppc = ppc_ref[0]
    ss_f = jnp.sum(img * img, axis=1, keepdims=True) + jnp.sum(pc * pc, axis=1, keepdims=True)
    ss_p = jnp.sum(pimg * pimg, axis=1, keepdims=True) + jnp.sum(ppc * ppc, axis=1, keepdims=True)
    den_f = jnp.maximum(jnp.sqrt(ss_f), 1e-8)
    den_p = jnp.maximum(jnp.sqrt(ss_p), 1e-8)
    fn_img = img / den_f
    fn_pc = pc / den_f
    pn_img = pimg / den_p
    pn_pc = ppc / den_p
    sims = _dot_t(fn_img, pn_img) + _dot_t(fn_pc, pn_pc)
    s_ref[0] = jnp.where(mask_ref[0], sims, NEG)


def _masked_sims(img, pc, pimg, ppc, mask_b):
    feat_spec = pl.BlockSpec((1, N, IMG_D), lambda b: (b, 0, 0))
    mat_spec = pl.BlockSpec((1, N, N), lambda b: (b, 0, 0))
    return pl.pallas_call(
        _sims_body,
        grid=(B,),
        in_specs=[feat_spec, feat_spec, feat_spec, feat_spec, mat_spec],
        out_specs=mat_spec,
        out_shape=jax.ShapeDtypeStruct((B, N, N), jnp.float32),
    )(img, pc, pimg, ppc, mask_b)


def _greedy_body(s_hbm, idx_hbm, s_v, vis_v, idx_v, rot_f, rot_i):
    info = plsc.get_sparse_core_info()
    wid = lax.axis_index("s") * info.num_cores + lax.axis_index("c")

    def rotmax_f(v):
        for k in (8, 4, 2, 1):
            rot_f[pl.ds(0, L)] = v
            rot_f[pl.ds(L, L)] = v
            v = jnp.maximum(v, rot_f[pl.ds(k, L)])
        return v

    def rotmin_i(v):
        for k in (8, 4, 2, 1):
            rot_i[pl.ds(0, L)] = v
            rot_i[pl.ds(L, L)] = v
            v = jnp.minimum(v, rot_i[pl.ds(k, L)])
        return v

    @pl.when(wid < B)
    def _():
        pltpu.sync_copy(s_hbm.at[wid], s_v)
        zeros = jnp.zeros((L,), jnp.float32)
        neg1 = jnp.full((L,), -1, jnp.int32)
        for c in range(NCHUNK):
            vis_v[pl.ds(c * L, L)] = zeros
            idx_v[pl.ds(c * L, L)] = neg1
        iota = lax.iota(jnp.int32, L)
        big = jnp.int32(32767)

        def body(i, carry):
            effs = []
            vm = jnp.full((L,), -3.0, jnp.float32)
            for c in range(NCHUNK):
                sv = s_v[i, pl.ds(c * L, L)]
                vis = vis_v[pl.ds(c * L, L)]
                eff = jnp.where(vis > 0.0, NEG, sv)
                effs.append(eff)
                vm = jnp.maximum(vm, eff)
            m_v = rotmax_f(vm)
            cand = jnp.full((L,), big, jnp.int32)
            for c in range(NCHUNK):
                cand = jnp.minimum(cand, jnp.where(effs[c] == m_v, iota + c * L, big))
            j_v = rotmin_i(cand)
            ok_v = m_v >= THRESH_VAL
            ibase = (i >> 4) << 4
            ioff = i & 15
            old_idx = idx_v[pl.ds(ibase, L)]
            idx_v[pl.ds(ibase, L)] = jnp.where(
                iota == ioff, jnp.where(ok_v, j_v, neg1), old_idx)
            for c in range(NCHUNK):
                vis = vis_v[pl.ds(c * L, L)]
                vis_v[pl.ds(c * L, L)] = jnp.where(
                    ((iota + c * L) == j_v) & ok_v, 1.0, vis)
            return carry

        lax.fori_loop(0, N, body, 0)
        pltpu.sync_copy(idx_v, idx_hbm.at[wid, 0])


def _greedy_match(s0):
    mesh = plsc.VectorSubcoreMesh(core_axis_name="c", subcore_axis_name="s")
    return pl.kernel(
        _greedy_body,
        out_type=jax.ShapeDtypeStruct((B, 1, N), jnp.int32),
        mesh=mesh,
        scratch_types=[
            pltpu.VMEM((N, N), jnp.float32),
            pltpu.VMEM((N,), jnp.float32),
            pltpu.VMEM((N,), jnp.int32),
            pltpu.VMEM((2 * L,), jnp.float32),
            pltpu.VMEM((2 * L,), jnp.int32),
        ],
    )(s0)


def _ln_rows(x, g, b):
    m = jnp.mean(x, axis=-1, keepdims=True)
    v = jnp.mean((x - m) ** 2, axis=-1, keepdims=True)
    return (x - m) / jnp.sqrt(v + 1e-5) * g + b


def _mlp_body(idx_ref, img_ref, pc_ref, pimg_ref, ppc_ref, pspat_ref,
              iW1_ref, ib1_ref, iW2_ref, ib2_ref, iW3_ref, ib3_ref, ig_ref, ibe_ref,
              pW1_ref, pb1_ref, pW2_ref, pb2_ref, pW3_ref, pb3_ref, pg_ref, pbe_ref,
              fW1_ref, fb1_ref, fW2_ref, fb2_ref, fg_ref, fbe_ref,
              vis_ref, nps_ref):
    idx = idx_ref[0]
    img = img_ref[0]
    pc = pc_ref[0]
    pimg = pimg_ref[0]
    ppc = ppc_ref[0]
    pspat = pspat_ref[0]
    jrow = lax.broadcasted_iota(jnp.int32, (N, N), 0)
    E = ((jrow == idx) & (idx >= 0)).astype(jnp.float32)

    def gather(prev):
        return lax.dot_general(E, prev, (((0,), (0,)), ((), ())), precision=HIGHEST,
                               preferred_element_type=jnp.float32)

    prev_img_g = gather(pimg)
    prev_pc_g = gather(ppc)
    nps_ref[0] = gather(pspat)

    iW1 = iW1_ref[...]
    h = jnp.maximum(_dot_t(img, iW1[:, :IMG_D]) + _dot_t(prev_img_g, iW1[:, IMG_D:])
                    + ib1_ref[...], 0.0)
    h = _dot_t(h, iW2_ref[...]) + ib2_ref[...]
    h = _dot_t(h, iW3_ref[...]) + ib3_ref[...]
    img_o = _ln_rows(h, ig_ref[...], ibe_ref[...])

    pW1 = pW1_ref[...]
    h = jnp.maximum(_dot_t(pc, pW1[:, :PC_D]) + _dot_t(prev_pc_g, pW1[:, PC_D:])
                    + pb1_ref[...], 0.0)
    h = _dot_t(h, pW2_ref[...]) + pb2_ref[...]
    h = _dot_t(h, pW3_ref[...]) + pb3_ref[...]
    pc_o = _ln_rows(h, pg_ref[...], pbe_ref[...])

    fW1 = fW1_ref[...]
    h = jnp.maximum(_dot_t(img_o, fW1[:, :VIS_D]) + _dot_t(pc_o, fW1[:, VIS_D:])
                    + fb1_ref[...], 0.0)
    h = _dot_t(h, fW2_ref[...]) + fb2_ref[...]
    vis_ref[0] = _ln_rows(h, fg_ref[...], fbe_ref[...])


def _fusion_mlp(idx3, img, pc, pimg, ppc, pspat, weights):
    feat_spec = pl.BlockSpec((1, N, IMG_D), lambda b: (b, 0, 0))
    spat_spec = pl.BlockSpec((1, N, 7), lambda b: (b, 0, 0))
    idx_spec = pl.BlockSpec((1, 1, N), lambda b: (b, 0, 0))
    full = lambda arr: pl.BlockSpec(arr.shape, lambda b: tuple(0 for _ in arr.shape))
    w_specs = [full(w) for w in weights]
    return pl.pallas_call(
        _mlp_body,
        grid=(B,),
        in_specs=[idx_spec, feat_spec, feat_spec, feat_spec, feat_spec, spat_spec] + w_specs,
        out_specs=[pl.BlockSpec((1, N, VIS_D), lambda b: (b, 0, 0)), spat_spec],
        out_shape=[jax.ShapeDtypeStruct((B, N, VIS_D), jnp.float32),
                   jax.ShapeDtypeStruct((B, N, 7), jnp.float32)],
    )(idx3, img, pc, pimg, ppc, pspat, *weights)


def kernel(image_feature, point_cloud_feature, prev_image_feature, prev_point_cloud_feature, rel_dist_mask, prev_spatial, img_W1, img_b1, img_W2, img_b2, img_W3, img_b3, img_g, img_be, pc_W1, pc_b1, pc_W2, pc_b2, pc_W3, pc_b3, pc_g, pc_be, fus_W1, fus_b1, fus_W2, fus_b2, fus_g, fus_be):
    s0 = _masked_sims(image_feature, point_cloud_feature,
                      prev_image_feature, prev_point_cloud_feature, rel_dist_mask)
    idx3 = _greedy_match(s0)
    r2 = lambda v: v.reshape(1, -1)
    weights = (img_W1, r2(img_b1), img_W2, r2(img_b2), img_W3, r2(img_b3), r2(img_g), r2(img_be),
               pc_W1, r2(pc_b1), pc_W2, r2(pc_b2), pc_W3, r2(pc_b3), r2(pc_g), r2(pc_be),
               fus_W1, r2(fus_b1), fus_W2, r2(fus_b2), r2(fus_g), r2(fus_be))
    vis, nps = _fusion_mlp(idx3, image_feature, point_cloud_feature,
                           prev_image_feature, prev_point_cloud_feature,
                           prev_spatial, weights)
    return (vis, nps)


# speedup vs baseline: 10.3270x; 1.2929x over previous
import functools

import jax
import jax.numpy as jnp
from jax import lax
from jax.experimental import pallas as pl
from jax.experimental.pallas import tpu as pltpu
from jax.experimental.pallas import tpu_sc as plsc

B, N = 4, 128
IMG_D = 256
PC_D = 256
VIS_D = 256
THRESH_VAL = 0.0
NEG = -2.0
HIGHEST = lax.Precision.HIGHEST
L = 16
NCHUNK = N // L


def _dot_t(x, w, precision=None):
    return lax.dot_general(x, w, (((1,), (1,)), ((), ())), precision=precision,
                           preferred_element_type=jnp.float32)


def _sims_body(img_ref, pc_ref, pimg_ref, ppc_ref, mask_ref, s_ref):
    img = img_ref[0]
    pc = pc_ref[0]
    pimg = pimg_ref[0]
    ppc = ppc_ref[0]
    ss_f = jnp.sum(img * img, axis=1, keepdims=True) + jnp.sum(pc * pc, axis=1, keepdims=True)
    ss_p = jnp.sum(pimg * pimg, axis=1, keepdims=True) + jnp.sum(ppc * ppc, axis=1, keepdims=True)
    den_f = jnp.maximum(jnp.sqrt(ss_f), 1e-8)
    den_p = jnp.maximum(jnp.sqrt(ss_p), 1e-8)
    fn_img = img / den_f
    fn_pc = pc / den_f
    pn_img = pimg / den_p
    pn_pc = ppc / den_p
    sims = _dot_t(fn_img, pn_img, HIGHEST) + _dot_t(fn_pc, pn_pc, HIGHEST)
    s_ref[0] = jnp.where(mask_ref[0], sims, NEG)


def _masked_sims(img, pc, pimg, ppc, mask_b):
    feat_spec = pl.BlockSpec((1, N, IMG_D), lambda b: (b, 0, 0))
    mat_spec = pl.BlockSpec((1, N, N), lambda b: (b, 0, 0))
    return pl.pallas_call(
        _sims_body,
        grid=(B,),
        in_specs=[feat_spec, feat_spec, feat_spec, feat_spec, mat_spec],
        out_specs=mat_spec,
        out_shape=jax.ShapeDtypeStruct((B, N, N), jnp.float32),
    )(img, pc, pimg, ppc, mask_b)


def _greedy_body(s_hbm, idx_hbm, s_v, vis_v, idx_v, rot_f, rot_i):
    info = plsc.get_sparse_core_info()
    wid = lax.axis_index("s") * info.num_cores + lax.axis_index("c")

    def rotmax_f(v):
        for k in (8, 4, 2, 1):
            rot_f[pl.ds(0, L)] = v
            rot_f[pl.ds(L, L)] = v
            v = jnp.maximum(v, rot_f[pl.ds(k, L)])
        return v

    def rotmin_i(v):
        for k in (8, 4, 2, 1):
            rot_i[pl.ds(0, L)] = v
            rot_i[pl.ds(L, L)] = v
            v = jnp.minimum(v, rot_i[pl.ds(k, L)])
        return v

    @pl.when(wid < B)
    def _():
        pltpu.sync_copy(s_hbm.at[wid], s_v)
        zeros = jnp.zeros((L,), jnp.float32)
        neg1 = jnp.full((L,), -1, jnp.int32)
        for c in range(NCHUNK):
            vis_v[pl.ds(c * L, L)] = zeros
            idx_v[pl.ds(c * L, L)] = neg1
        iota = lax.iota(jnp.int32, L)
        big = jnp.int32(32767)

        def body(i, carry):
            effs = []
            vm = jnp.full((L,), -3.0, jnp.float32)
            for c in range(NCHUNK):
                sv = s_v[i, pl.ds(c * L, L)]
                vis = vis_v[pl.ds(c * L, L)]
                eff = jnp.where(vis > 0.0, NEG, sv)
                effs.append(eff)
                vm = jnp.maximum(vm, eff)
            m_v = rotmax_f(vm)
            cand = jnp.full((L,), big, jnp.int32)
            for c in range(NCHUNK):
                cand = jnp.minimum(cand, jnp.where(effs[c] == m_v, iota + c * L, big))
            j_v = rotmin_i(cand)
            ok_v = m_v >= THRESH_VAL
            ibase = (i >> 4) << 4
            ioff = i & 15
            old_idx = idx_v[pl.ds(ibase, L)]
            idx_v[pl.ds(ibase, L)] = jnp.where(
                iota == ioff, jnp.where(ok_v, j_v, neg1), old_idx)
            for c in range(NCHUNK):
                vis = vis_v[pl.ds(c * L, L)]
                vis_v[pl.ds(c * L, L)] = jnp.where(
                    ((iota + c * L) == j_v) & ok_v, 1.0, vis)
            return carry

        lax.fori_loop(0, N, body, 0)
        pltpu.sync_copy(idx_v, idx_hbm.at[wid, 0])


def _greedy_match(s0):
    mesh = plsc.VectorSubcoreMesh(core_axis_name="c", subcore_axis_name="s")
    return pl.kernel(
        _greedy_body,
        out_type=jax.ShapeDtypeStruct((B, 1, N), jnp.int32),
        mesh=mesh,
        scratch_types=[
            pltpu.VMEM((N, N), jnp.float32),
            pltpu.VMEM((N,), jnp.float32),
            pltpu.VMEM((N,), jnp.int32),
            pltpu.VMEM((2 * L,), jnp.float32),
            pltpu.VMEM((2 * L,), jnp.int32),
        ],
    )(s0)


def _ln_rows(x, g, b):
    m = jnp.mean(x, axis=-1, keepdims=True)
    v = jnp.mean((x - m) ** 2, axis=-1, keepdims=True)
    return (x - m) / jnp.sqrt(v + 1e-5) * g + b


def _mlp_body(idx_ref, img_ref, pc_ref, pimg_ref, ppc_ref, pspat_ref,
              iW1_ref, ib1_ref, iW2_ref, ib2_ref, iW3_ref, ib3_ref, ig_ref, ibe_ref,
              pW1_ref, pb1_ref, pW2_ref, pb2_ref, pW3_ref, pb3_ref, pg_ref, pbe_ref,
              fW1_ref, fb1_ref, fW2_ref, fb2_ref, fg_ref, fbe_ref,
              vis_ref, nps_ref):
    idx = idx_ref[0]
    img = img_ref[0]
    pc = pc_ref[0]
    pimg = pimg_ref[0]
    ppc = ppc_ref[0]
    pspat = pspat_ref[0]
    jrow = lax.broadcasted_iota(jnp.int32, (N, N), 0)
    E = ((jrow == idx) & (idx >= 0)).astype(jnp.float32)

    def gather(prev):
        return lax.dot_general(E, prev, (((0,), (0,)), ((), ())), precision=HIGHEST,
                               preferred_element_type=jnp.float32)

    prev_img_g = gather(pimg)
    prev_pc_g = gather(ppc)
    nps_ref[0] = gather(pspat)

    iW1 = iW1_ref[...]
    h = jnp.maximum(_dot_t(img, iW1[:, :IMG_D]) + _dot_t(prev_img_g, iW1[:, IMG_D:])
                    + ib1_ref[...], 0.0)
    h = _dot_t(h, iW2_ref[...]) + ib2_ref[...]
    h = _dot_t(h, iW3_ref[...]) + ib3_ref[...]
    img_o = _ln_rows(h, ig_ref[...], ibe_ref[...])

    pW1 = pW1_ref[...]
    h = jnp.maximum(_dot_t(pc, pW1[:, :PC_D]) + _dot_t(prev_pc_g, pW1[:, PC_D:])
                    + pb1_ref[...], 0.0)
    h = _dot_t(h, pW2_ref[...]) + pb2_ref[...]
    h = _dot_t(h, pW3_ref[...]) + pb3_ref[...]
    pc_o = _ln_rows(h, pg_ref[...], pbe_ref[...])

    fW1 = fW1_ref[...]
    h = jnp.maximum(_dot_t(img_o, fW1[:, :VIS_D]) + _dot_t(pc_o, fW1[:, VIS_D:])
                    + fb1_ref[...], 0.0)
    h = _dot_t(h, fW2_ref[...]) + fb2_ref[...]
    vis_ref[0] = _ln_rows(h, fg_ref[...], fbe_ref[...])


def _fusion_mlp(idx3, img, pc, pimg, ppc, pspat, weights):
    feat_spec = pl.BlockSpec((1, N, IMG_D), lambda b: (b, 0, 0))
    spat_spec = pl.BlockSpec((1, N, 7), lambda b: (b, 0, 0))
    idx_spec = pl.BlockSpec((1, 1, N), lambda b: (b, 0, 0))
    full = lambda arr: pl.BlockSpec(arr.shape, lambda b: tuple(0 for _ in arr.shape))
    w_specs = [full(w) for w in weights]
    return pl.pallas_call(
        _mlp_body,
        grid=(B,),
        in_specs=[idx_spec, feat_spec, feat_spec, feat_spec, feat_spec, spat_spec] + w_specs,
        out_specs=[pl.BlockSpec((1, N, VIS_D), lambda b: (b, 0, 0)), spat_spec],
        out_shape=[jax.ShapeDtypeStruct((B, N, VIS_D), jnp.float32),
                   jax.ShapeDtypeStruct((B, N, 7), jnp.float32)],
    )(idx3, img, pc, pimg, ppc, pspat, *weights)


def kernel(image_feature, point_cloud_feature, prev_image_feature, prev_point_cloud_feature, rel_dist_mask, prev_spatial, img_W1, img_b1, img_W2, img_b2, img_W3, img_b3, img_g, img_be, pc_W1, pc_b1, pc_W2, pc_b2, pc_W3, pc_b3, pc_g, pc_be, fus_W1, fus_b1, fus_W2, fus_b2, fus_g, fus_be):
    s0 = _masked_sims(image_feature, point_cloud_feature,
                      prev_image_feature, prev_point_cloud_feature, rel_dist_mask)
    idx3 = _greedy_match(s0)
    r2 = lambda v: v.reshape(1, -1)
    weights = (img_W1, r2(img_b1), img_W2, r2(img_b2), img_W3, r2(img_b3), r2(img_g), r2(img_be),
               pc_W1, r2(pc_b1), pc_W2, r2(pc_b2), pc_W3, r2(pc_b3), r2(pc_g), r2(pc_be),
               fus_W1, r2(fus_b1), fus_W2, r2(fus_b2), r2(fus_g), r2(fus_be))
    vis, nps = _fusion_mlp(idx3, image_feature, point_cloud_feature,
                           prev_image_feature, prev_point_cloud_feature,
                           prev_spatial, weights)
    return (vis, nps)


# speedup vs baseline: 11.1324x; 1.0780x over previous
import functools

import jax
import jax.numpy as jnp
from jax import lax
from jax.experimental import pallas as pl
from jax.experimental.pallas import tpu as pltpu
from jax.experimental.pallas import tpu_sc as plsc

B, N = 4, 128
IMG_D = 256
PC_D = 256
VIS_D = 256
THRESH_VAL = 0.0
NEG = -2.0
HIGHEST = lax.Precision.HIGHEST
L = 16
NCHUNK = N // L


def _dot_t(x, w, precision=None):
    return lax.dot_general(x, w, (((1,), (1,)), ((), ())), precision=precision,
                           preferred_element_type=jnp.float32)


def _sims_body(img_ref, pc_ref, pimg_ref, ppc_ref, mask_ref, s_ref):
    img = img_ref[0]
    pc = pc_ref[0]
    pimg = pimg_ref[0]
    ppc = ppc_ref[0]
    ss_f = jnp.sum(img * img, axis=1, keepdims=True) + jnp.sum(pc * pc, axis=1, keepdims=True)
    ss_p = jnp.sum(pimg * pimg, axis=1, keepdims=True) + jnp.sum(ppc * ppc, axis=1, keepdims=True)
    den_f = jnp.maximum(jnp.sqrt(ss_f), 1e-8)
    den_p = jnp.maximum(jnp.sqrt(ss_p), 1e-8)
    fn_img = img / den_f
    fn_pc = pc / den_f
    pn_img = pimg / den_p
    pn_pc = ppc / den_p
    sims = _dot_t(fn_img, pn_img, HIGHEST) + _dot_t(fn_pc, pn_pc, HIGHEST)
    s_ref[0] = jnp.where(mask_ref[0], sims, NEG)


def _masked_sims(img, pc, pimg, ppc, mask_b):
    feat_spec = pl.BlockSpec((1, N, IMG_D), lambda b: (b, 0, 0))
    mat_spec = pl.BlockSpec((1, N, N), lambda b: (b, 0, 0))
    return pl.pallas_call(
        _sims_body,
        grid=(B,),
        in_specs=[feat_spec, feat_spec, feat_spec, feat_spec, mat_spec],
        out_specs=mat_spec,
        out_shape=jax.ShapeDtypeStruct((B, N, N), jnp.float32),
    )(img, pc, pimg, ppc, mask_b)


def _premlp_body(img_ref, pc_ref, pimg_ref, ppc_ref, iW1_ref, pW1_ref,
                 pre_img_ref, pre_pc_ref, z_img_ref, z_pc_ref):
    iW1 = iW1_ref[...]
    pW1 = pW1_ref[...]
    pre_img_ref[0] = _dot_t(img_ref[0], iW1[:, :IMG_D])
    pre_pc_ref[0] = _dot_t(pc_ref[0], pW1[:, :PC_D])
    z_img_ref[0] = _dot_t(pimg_ref[0], iW1[:, IMG_D:])
    z_pc_ref[0] = _dot_t(ppc_ref[0], pW1[:, PC_D:])


def _premlp(img, pc, pimg, ppc, iW1, pW1):
    feat_spec = pl.BlockSpec((1, N, IMG_D), lambda b: (b, 0, 0))
    full = lambda arr: pl.BlockSpec(arr.shape, lambda b: tuple(0 for _ in arr.shape))
    return pl.pallas_call(
        _premlp_body,
        grid=(B,),
        in_specs=[feat_spec, feat_spec, feat_spec, feat_spec, full(iW1), full(pW1)],
        out_specs=[feat_spec] * 4,
        out_shape=[jax.ShapeDtypeStruct((B, N, IMG_D), jnp.float32)] * 4,
    )(img, pc, pimg, ppc, iW1, pW1)


def _greedy_body(s_hbm, idx_hbm, s_v, idx_v, rot_f, rot_i):
    info = plsc.get_sparse_core_info()
    wid = lax.axis_index("s") * info.num_cores + lax.axis_index("c")

    def rotmax_f(v):
        for k in (8, 4, 2, 1):
            rot_f[pl.ds(0, L)] = v
            rot_f[pl.ds(L, L)] = v
            v = jnp.maximum(v, rot_f[pl.ds(k, L)])
        return v

    def rotmin_i(v):
        for k in (8, 4, 2, 1):
            rot_i[pl.ds(0, L)] = v
            rot_i[pl.ds(L, L)] = v
            v = jnp.minimum(v, rot_i[pl.ds(k, L)])
        return v

    @pl.when(wid < B)
    def _():
        pltpu.sync_copy(s_hbm.at[wid], s_v)
        iota = lax.iota(jnp.int32, L)
        big = jnp.int32(32767)
        zeros = jnp.zeros((L,), jnp.float32)
        neg1 = jnp.full((L,), -1, jnp.int32)

        def body(i, carry):
            vis = carry[:NCHUNK]
            idxs = carry[NCHUNK:]
            vm = jnp.full((L,), -3.0, jnp.float32)
            cidx = jnp.full((L,), big, jnp.int32)
            for c in range(NCHUNK):
                sv = s_v[i, pl.ds(c * L, L)]
                eff = jnp.where(vis[c] > 0.0, NEG, sv)
                upd = eff > vm
                cidx = jnp.where(upd, iota + c * L, cidx)
                vm = jnp.maximum(vm, eff)
            for k in (8, 4, 2, 1):
                rot_f[pl.ds(0, L)] = vm
                rot_f[pl.ds(L, L)] = vm
                rot_i[pl.ds(0, L)] = cidx
                rot_i[pl.ds(L, L)] = cidx
                rvm = rot_f[pl.ds(k, L)]
                rci = rot_i[pl.ds(k, L)]
                take = (rvm > vm) | ((rvm == vm) & (rci < cidx))
                vm = jnp.where(take, rvm, vm)
                cidx = jnp.where(take, rci, cidx)
            ok_v = vm >= THRESH_VAL
            j_sel = jnp.where(ok_v, cidx, big)
            new_vis = tuple(
                jnp.where((iota + c * L) == j_sel, 1.0, vis[c])
                for c in range(NCHUNK))
            new_idxs = tuple(
                jnp.where((iota + c * L) == i,
                          jnp.where(ok_v, cidx, neg1), idxs[c])
                for c in range(NCHUNK))
            return new_vis + new_idxs

        init = (zeros,) * NCHUNK + (neg1,) * NCHUNK
        final = lax.fori_loop(0, N, body, init)
        for c in range(NCHUNK):
            idx_v[pl.ds(c * L, L)] = final[NCHUNK + c]
        pltpu.sync_copy(idx_v, idx_hbm.at[wid, 0])


def _greedy_match(s0):
    mesh = plsc.VectorSubcoreMesh(core_axis_name="c", subcore_axis_name="s")
    return pl.kernel(
        _greedy_body,
        out_type=jax.ShapeDtypeStruct((B, 1, N), jnp.int32),
        mesh=mesh,
        scratch_types=[
            pltpu.VMEM((N, N), jnp.float32),
            pltpu.VMEM((N,), jnp.int32),
            pltpu.VMEM((2 * L,), jnp.float32),
            pltpu.VMEM((2 * L,), jnp.int32),
        ],
    )(s0)


def _ln_rows(x, g, b):
    m = jnp.mean(x, axis=-1, keepdims=True)
    v = jnp.mean((x - m) ** 2, axis=-1, keepdims=True)
    return (x - m) / jnp.sqrt(v + 1e-5) * g + b


def _mlp_body(idx_ref, pre_img_ref, pre_pc_ref, z_img_ref, z_pc_ref, pspat_ref,
              ib1_ref, iW2_ref, ib2_ref, iW3_ref, ib3_ref, ig_ref, ibe_ref,
              pb1_ref, pW2_ref, pb2_ref, pW3_ref, pb3_ref, pg_ref, pbe_ref,
              fW1_ref, fb1_ref, fW2_ref, fb2_ref, fg_ref, fbe_ref,
              vis_ref, nps_ref):
    idx = idx_ref[0]
    pspat = pspat_ref[0]
    jrow = lax.broadcasted_iota(jnp.int32, (N, N), 0)
    E = ((jrow == idx) & (idx >= 0)).astype(jnp.float32)

    def gather(prev):
        return lax.dot_general(E, prev, (((0,), (0,)), ((), ())), precision=HIGHEST,
                               preferred_element_type=jnp.float32)

    nps_ref[0] = gather(pspat)

    h = jnp.maximum(pre_img_ref[0] + gather(z_img_ref[0]) + ib1_ref[...], 0.0)
    h = _dot_t(h, iW2_ref[...]) + ib2_ref[...]
    h = _dot_t(h, iW3_ref[...]) + ib3_ref[...]
    img_o = _ln_rows(h, ig_ref[...], ibe_ref[...])

    h = jnp.maximum(pre_pc_ref[0] + gather(z_pc_ref[0]) + pb1_ref[...], 0.0)
    h = _dot_t(h, pW2_ref[...]) + pb2_ref[...]
    h = _dot_t(h, pW3_ref[...]) + pb3_ref[...]
    pc_o = _ln_rows(h, pg_ref[...], pbe_ref[...])

    fW1 = fW1_ref[...]
    h = jnp.maximum(_dot_t(img_o, fW1[:, :VIS_D]) + _dot_t(pc_o, fW1[:, VIS_D:])
                    + fb1_ref[...], 0.0)
    h = _dot_t(h, fW2_ref[...]) + fb2_ref[...]
    vis_ref[0] = _ln_rows(h, fg_ref[...], fbe_ref[...])


def _fusion_mlp(idx3, pre_img, pre_pc, z_img, z_pc, pspat, weights):
    feat_spec = pl.BlockSpec((1, N, IMG_D), lambda b: (b, 0, 0))
    spat_spec = pl.BlockSpec((1, N, 7), lambda b: (b, 0, 0))
    idx_spec = pl.BlockSpec((1, 1, N), lambda b: (b, 0, 0))
    full = lambda arr: pl.BlockSpec(arr.shape, lambda b: tuple(0 for _ in arr.shape))
    w_specs = [full(w) for w in weights]
    return pl.pallas_call(
        _mlp_body,
        grid=(B,),
        in_specs=[idx_spec, feat_spec, feat_spec, feat_spec, feat_spec, spat_spec] + w_specs,
        out_specs=[pl.BlockSpec((1, N, VIS_D), lambda b: (b, 0, 0)), spat_spec],
        out_shape=[jax.ShapeDtypeStruct((B, N, VIS_D), jnp.float32),
                   jax.ShapeDtypeStruct((B, N, 7), jnp.float32)],
    )(idx3, pre_img, pre_pc, z_img, z_pc, pspat, *weights)


def kernel(image_feature, point_cloud_feature, prev_image_feature, prev_point_cloud_feature, rel_dist_mask, prev_spatial, img_W1, img_b1, img_W2, img_b2, img_W3, img_b3, img_g, img_be, pc_W1, pc_b1, pc_W2, pc_b2, pc_W3, pc_b3, pc_g, pc_be, fus_W1, fus_b1, fus_W2, fus_b2, fus_g, fus_be):
    s0 = _masked_sims(image_feature, point_cloud_feature,
                      prev_image_feature, prev_point_cloud_feature, rel_dist_mask)
    idx3 = _greedy_match(s0)
    pre_img, pre_pc, z_img, z_pc = _premlp(image_feature, point_cloud_feature,
                                           prev_image_feature, prev_point_cloud_feature,
                                           img_W1, pc_W1)
    r2 = lambda v: v.reshape(1, -1)
    weights = (r2(img_b1), img_W2, r2(img_b2), img_W3, r2(img_b3), r2(img_g), r2(img_be),
               r2(pc_b1), pc_W2, r2(pc_b2), pc_W3, r2(pc_b3), r2(pc_g), r2(pc_be),
               fus_W1, r2(fus_b1), fus_W2, r2(fus_b2), r2(fus_g), r2(fus_be))
    vis, nps = _fusion_mlp(idx3, pre_img, pre_pc, z_img, z_pc,
                           prev_spatial, weights)
    return (vis, nps)
